# initial kernel scaffold (unmeasured)
import jax
import jax.numpy as jnp
from jax import lax
from jax.experimental import pallas as pl
from jax.experimental.pallas import tpu as pltpu

N_DEV = 8
HQ = 8
DH = 128
SQ = 256
D = 1024
SCALE = 0.08838834764831843
N_ROUNDS = 3


def kernel(x, Wq, Wo, K_ext, V_ext):
    skv = K_ext.shape[1]
    x2 = x.reshape(SQ, D)
    k2 = K_ext.reshape(skv, HQ * DH)
    v2 = V_ext.reshape(skv, HQ * DH)

    def body(x_ref, wq_ref, wo_ref, k_ref, v_ref, out_ref,
             outc, statc, send_sems, recv_sems):
        my = lax.axis_index("i")

        barrier = pltpu.get_barrier_semaphore()
        for r in range(N_ROUNDS):
            pl.semaphore_signal(
                barrier, inc=1,
                device_id=(my ^ (1 << r),),
                device_id_type=pl.DeviceIdType.MESH,
            )
        pl.semaphore_wait(barrier, N_ROUNDS)

        xq = x_ref[:, :].astype(jnp.bfloat16)
        wq = wq_ref[:, :].astype(jnp.bfloat16)
        q = lax.dot_general(xq, wq, (((1,), (0,)), ((), ())),
                            preferred_element_type=jnp.float32)

        for h in range(HQ):
            qh = q[:, h * DH:(h + 1) * DH].astype(jnp.bfloat16)
            kh = k_ref[:, h * DH:(h + 1) * DH].astype(jnp.bfloat16)
            vh = v_ref[:, h * DH:(h + 1) * DH].astype(jnp.bfloat16)
            st = lax.dot_general(kh, qh, (((1,), (1,)), ((), ())),
                                 preferred_element_type=jnp.float32)
            st = st * SCALE
            mt = jnp.max(st, axis=0, keepdims=True)
            pt = jnp.exp(st - mt)
            lt = jnp.sum(pt, axis=0, keepdims=True)
            ot = lax.dot_general(vh, pt.astype(jnp.bfloat16),
                                 (((0,), (0,)), ((), ())),
                                 preferred_element_type=jnp.float32)
            outc[0, h] = ot
            statc[0, h:h + 1, :] = mt
            statc[0, HQ + h:HQ + h + 1, :] = lt

        for r in range(N_ROUNDS):
            partner = my ^ (1 << r)
            s_out = pltpu.make_async_remote_copy(
                src_ref=outc.at[2 * r],
                dst_ref=outc.at[2 * r + 1],
                send_sem=send_sems.at[r, 0],
                recv_sem=recv_sems.at[r, 0],
                device_id=(partner,),
                device_id_type=pl.DeviceIdType.MESH,
            )
            s_st = pltpu.make_async_remote_copy(
                src_ref=statc.at[2 * r],
                dst_ref=statc.at[2 * r + 1],
                send_sem=send_sems.at[r, 1],
                recv_sem=recv_sems.at[r, 1],
                device_id=(partner,),
                device_id_type=pl.DeviceIdType.MESH,
            )
            s_out.start()
            s_st.start()
            s_out.wait()
            s_st.wait()

            m1 = statc[2 * r, 0:HQ, :]
            l1 = statc[2 * r, HQ:2 * HQ, :]
            m2 = statc[2 * r + 1, 0:HQ, :]
            l2 = statc[2 * r + 1, HQ:2 * HQ, :]
            mm = jnp.maximum(m1, m2)
            w1 = jnp.exp(m1 - mm)
            w2 = jnp.exp(m2 - mm)
            lm = w1 * l1 + w2 * l2
            om = (w1[:, None, :] * outc[2 * r]
                  + w2[:, None, :] * outc[2 * r + 1])

            if r < N_ROUNDS - 1:
                outc[2 * (r + 1)] = om
                statc[2 * (r + 1), 0:HQ, :] = mm
                statc[2 * (r + 1), HQ:2 * HQ, :] = lm
            else:
                attn = om / lm[:, None, :]
                acc = jnp.zeros((SQ, D), jnp.float32)
                for h in range(HQ):
                    ah = attn[h].astype(jnp.bfloat16)
                    woh = wo_ref[h * DH:(h + 1) * DH, :].astype(jnp.bfloat16)
                    acc = acc + lax.dot_general(
                        ah, woh, (((0,), (0,)), ((), ())),
                        preferred_element_type=jnp.float32)
                out_ref[:, :] = acc

    out = pl.pallas_call(
        body,
        out_shape=jax.ShapeDtypeStruct((SQ, D), jnp.float32),
        in_specs=[pl.BlockSpec(memory_space=pltpu.VMEM)] * 5,
        out_specs=pl.BlockSpec(memory_space=pltpu.VMEM),
        scratch_shapes=[
            pltpu.VMEM((2 * N_ROUNDS, HQ, DH, SQ), jnp.float32),
            pltpu.VMEM((2 * N_ROUNDS, 2 * HQ, SQ), jnp.float32),
            pltpu.SemaphoreType.DMA((N_ROUNDS, 2)),
            pltpu.SemaphoreType.DMA((N_ROUNDS, 2)),
        ],
        compiler_params=pltpu.CompilerParams(collective_id=0),
    )(x2, wq := Wq, wo := Wo, k2, v2)
    return out.reshape(1, SQ, D)


# baseline (device time: 109931 ns/iter reference)
import jax
import jax.numpy as jnp
from jax import lax
from jax.experimental import pallas as pl
from jax.experimental.pallas import tpu as pltpu

N_DEV = 8
HQ = 8
DH = 128
SQ = 256
D = 1024
SCALE = 0.08838834764831843
N_ROUNDS = 3


def kernel(x, Wq, Wo, K_ext, V_ext):
    skv = K_ext.shape[1]
    x2 = x.reshape(SQ, D)
    k2 = K_ext.reshape(skv, HQ * DH)
    v2 = V_ext.reshape(skv, HQ * DH)

    def body(x_ref, wq_ref, wo_ref, k_ref, v_ref, out_ref,
             outc, statc, send_sems, recv_sems):
        my = lax.axis_index("i")

        barrier = pltpu.get_barrier_semaphore()
        for r in range(N_ROUNDS):
            pl.semaphore_signal(
                barrier, inc=1,
                device_id=(my ^ (1 << r),),
                device_id_type=pl.DeviceIdType.MESH,
            )
        pl.semaphore_wait(barrier, N_ROUNDS)

        xq = x_ref[:, :].astype(jnp.bfloat16)
        wq = wq_ref[:, :].astype(jnp.bfloat16)
        q = lax.dot_general(xq, wq, (((1,), (0,)), ((), ())),
                            preferred_element_type=jnp.float32)

        for h in range(HQ):
            qh = q[:, h * DH:(h + 1) * DH].astype(jnp.bfloat16)
            kh = k_ref[:, h * DH:(h + 1) * DH].astype(jnp.bfloat16)
            vh = v_ref[:, h * DH:(h + 1) * DH].astype(jnp.bfloat16)
            st = lax.dot_general(kh, qh, (((1,), (1,)), ((), ())),
                                 preferred_element_type=jnp.float32)
            st = st * SCALE
            mt = jnp.max(st, axis=0, keepdims=True)
            pt = jnp.exp(st - mt)
            lt = jnp.sum(pt, axis=0, keepdims=True)
            ot = lax.dot_general(vh, pt.astype(jnp.bfloat16),
                                 (((0,), (0,)), ((), ())),
                                 preferred_element_type=jnp.float32)
            outc[0, h] = ot
            statc[0, h:h + 1, :] = mt
            statc[0, HQ + h:HQ + h + 1, :] = lt

        for r in range(N_ROUNDS):
            partner = my ^ (1 << r)
            s_out = pltpu.make_async_remote_copy(
                src_ref=outc.at[2 * r],
                dst_ref=outc.at[2 * r + 1],
                send_sem=send_sems.at[r, 0],
                recv_sem=recv_sems.at[r, 0],
                device_id=(partner,),
                device_id_type=pl.DeviceIdType.MESH,
            )
            s_st = pltpu.make_async_remote_copy(
                src_ref=statc.at[2 * r],
                dst_ref=statc.at[2 * r + 1],
                send_sem=send_sems.at[r, 1],
                recv_sem=recv_sems.at[r, 1],
                device_id=(partner,),
                device_id_type=pl.DeviceIdType.MESH,
            )
            s_out.start()
            s_st.start()
            s_out.wait()
            s_st.wait()

            m1 = statc[2 * r, 0:HQ, :]
            l1 = statc[2 * r, HQ:2 * HQ, :]
            m2 = statc[2 * r + 1, 0:HQ, :]
            l2 = statc[2 * r + 1, HQ:2 * HQ, :]
            mm = jnp.maximum(m1, m2)
            w1 = jnp.exp(m1 - mm)
            w2 = jnp.exp(m2 - mm)
            lm = w1 * l1 + w2 * l2
            om = (w1[:, None, :] * outc[2 * r]
                  + w2[:, None, :] * outc[2 * r + 1])

            if r < N_ROUNDS - 1:
                outc[2 * (r + 1)] = om
                statc[2 * (r + 1), 0:HQ, :] = mm
                statc[2 * (r + 1), HQ:2 * HQ, :] = lm
            else:
                attn = om / lm[:, None, :]
                acc = jnp.zeros((SQ, D), jnp.float32)
                for h in range(HQ):
                    ah = attn[h].astype(jnp.bfloat16)
                    woh = wo_ref[h * DH:(h + 1) * DH, :].astype(jnp.bfloat16)
                    acc = acc + lax.dot_general(
                        ah, woh, (((0,), (0,)), ((), ())),
                        preferred_element_type=jnp.float32)
                out_ref[:, :] = acc

    out = pl.pallas_call(
        body,
        out_shape=jax.ShapeDtypeStruct((SQ, D), jnp.float32),
        in_specs=[pl.BlockSpec(memory_space=pltpu.VMEM)] * 5,
        out_specs=pl.BlockSpec(memory_space=pltpu.VMEM),
        scratch_shapes=[
            pltpu.VMEM((2 * N_ROUNDS, HQ, DH, SQ), jnp.float32),
            pltpu.VMEM((2 * N_ROUNDS, 2 * HQ, SQ), jnp.float32),
            pltpu.SemaphoreType.DMA((N_ROUNDS, 2)),
            pltpu.SemaphoreType.DMA((N_ROUNDS, 2)),
        ],
        compiler_params=pltpu.CompilerParams(
            collective_id=0,
            vmem_limit_bytes=100 * 1024 * 1024,
        ),
    )(x2, Wq, Wo, k2, v2)
    return out.reshape(1, SQ, D)


# device time: 86210 ns/iter; 1.2752x vs baseline; 1.2752x over previous
import jax
import jax.numpy as jnp
from jax import lax
from jax.experimental import pallas as pl
from jax.experimental.pallas import tpu as pltpu

N_DEV = 8
HQ = 8
DH = 128
SQ = 256
D = 1024
SCALE = 0.08838834764831843
N_ROUNDS = 3


def kernel(x, Wq, Wo, K_ext, V_ext):
    skv = K_ext.shape[1]
    x2 = x.reshape(SQ, D)
    k3 = K_ext.reshape(skv, HQ, DH)
    v3 = V_ext.reshape(skv, HQ, DH)

    def body(x_ref, wq_ref, wo_ref, k_ref, v_ref, out_ref,
             kbuf, vbuf, outc, statc, copy_sems, send_sems, recv_sems):
        my = lax.axis_index("i")

        barrier = pltpu.get_barrier_semaphore()
        for r in range(N_ROUNDS):
            pl.semaphore_signal(
                barrier, inc=1,
                device_id=(my ^ (1 << r),),
                device_id_type=pl.DeviceIdType.MESH,
            )
        pl.semaphore_wait(barrier, N_ROUNDS)

        def start_head_dma(h, slot):
            pltpu.make_async_copy(
                k_ref.at[:, h, :], kbuf.at[slot], copy_sems.at[slot, 0]
            ).start()
            pltpu.make_async_copy(
                v_ref.at[:, h, :], vbuf.at[slot], copy_sems.at[slot, 1]
            ).start()

        def wait_head_dma(h, slot):
            pltpu.make_async_copy(
                k_ref.at[:, h, :], kbuf.at[slot], copy_sems.at[slot, 0]
            ).wait()
            pltpu.make_async_copy(
                v_ref.at[:, h, :], vbuf.at[slot], copy_sems.at[slot, 1]
            ).wait()

        xq = x_ref[:, :].astype(jnp.bfloat16)
        wq = wq_ref[:, :].astype(jnp.bfloat16)
        q = lax.dot_general(xq, wq, (((1,), (0,)), ((), ())),
                            preferred_element_type=jnp.float32)

        start_head_dma(0, 0)
        start_head_dma(1, 1)
        for h in range(HQ):
            slot = h % 2
            wait_head_dma(h, slot)
            qh = q[:, h * DH:(h + 1) * DH].astype(jnp.bfloat16)
            kh = kbuf[slot].astype(jnp.bfloat16)
            vh = vbuf[slot].astype(jnp.bfloat16)
            st = lax.dot_general(kh, qh, (((1,), (1,)), ((), ())),
                                 preferred_element_type=jnp.float32)
            st = st * SCALE
            mt = jnp.max(st, axis=0, keepdims=True)
            pt = jnp.exp(st - mt)
            lt = jnp.sum(pt, axis=0, keepdims=True)
            ot = lax.dot_general(vh, pt.astype(jnp.bfloat16),
                                 (((0,), (0,)), ((), ())),
                                 preferred_element_type=jnp.float32)
            if h + 2 < HQ:
                start_head_dma(h + 2, slot)
            outc[0, h] = ot.astype(jnp.bfloat16)
            statc[0, h:h + 1, :] = mt
            statc[0, HQ + h:HQ + h + 1, :] = lt

        for r in range(N_ROUNDS):
            partner = my ^ (1 << r)
            s_out = pltpu.make_async_remote_copy(
                src_ref=outc.at[2 * r],
                dst_ref=outc.at[2 * r + 1],
                send_sem=send_sems.at[r, 0],
                recv_sem=recv_sems.at[r, 0],
                device_id=(partner,),
                device_id_type=pl.DeviceIdType.MESH,
            )
            s_st = pltpu.make_async_remote_copy(
                src_ref=statc.at[2 * r],
                dst_ref=statc.at[2 * r + 1],
                send_sem=send_sems.at[r, 1],
                recv_sem=recv_sems.at[r, 1],
                device_id=(partner,),
                device_id_type=pl.DeviceIdType.MESH,
            )
            s_out.start()
            s_st.start()
            s_out.wait()
            s_st.wait()

            m1 = statc[2 * r, 0:HQ, :]
            l1 = statc[2 * r, HQ:2 * HQ, :]
            m2 = statc[2 * r + 1, 0:HQ, :]
            l2 = statc[2 * r + 1, HQ:2 * HQ, :]
            mm = jnp.maximum(m1, m2)
            w1 = jnp.exp(m1 - mm)
            w2 = jnp.exp(m2 - mm)
            lm = w1 * l1 + w2 * l2
            om = (w1[:, None, :] * outc[2 * r].astype(jnp.float32)
                  + w2[:, None, :] * outc[2 * r + 1].astype(jnp.float32))

            if r < N_ROUNDS - 1:
                outc[2 * (r + 1)] = om.astype(jnp.bfloat16)
                statc[2 * (r + 1), 0:HQ, :] = mm
                statc[2 * (r + 1), HQ:2 * HQ, :] = lm
            else:
                attn = om / lm[:, None, :]
                acc = jnp.zeros((SQ, D), jnp.float32)
                for h in range(HQ):
                    ah = attn[h].astype(jnp.bfloat16)
                    woh = wo_ref[h * DH:(h + 1) * DH, :].astype(jnp.bfloat16)
                    acc = acc + lax.dot_general(
                        ah, woh, (((0,), (0,)), ((), ())),
                        preferred_element_type=jnp.float32)
                out_ref[:, :] = acc

    out = pl.pallas_call(
        body,
        out_shape=jax.ShapeDtypeStruct((SQ, D), jnp.float32),
        in_specs=[pl.BlockSpec(memory_space=pltpu.VMEM)] * 5,
        out_specs=pl.BlockSpec(memory_space=pltpu.VMEM),
        scratch_shapes=[
            pltpu.VMEM((2, 4096, DH), jnp.float32),
            pltpu.VMEM((2, 4096, DH), jnp.float32),
            pltpu.VMEM((2 * N_ROUNDS, HQ, DH, SQ), jnp.bfloat16),
            pltpu.VMEM((2 * N_ROUNDS, 2 * HQ, SQ), jnp.float32),
            pltpu.SemaphoreType.DMA((2, 2)),
            pltpu.SemaphoreType.DMA((N_ROUNDS, 2)),
            pltpu.SemaphoreType.DMA((N_ROUNDS, 2)),
        ],
        compiler_params=pltpu.CompilerParams(
            collective_id=0,
            vmem_limit_bytes=100 * 1024 * 1024,
        ),
    )(x2, Wq, Wo, k3, v3)
    return out.reshape(1, SQ, D)


# device time: 60791 ns/iter; 1.8083x vs baseline; 1.4181x over previous
import jax
import jax.numpy as jnp
from jax import lax
from jax.experimental import pallas as pl
from jax.experimental.pallas import tpu as pltpu

N_DEV = 8
HQ = 8
DH = 128
SQ = 256
D = 1024
SCALE = 0.08838834764831843
N_ROUNDS = 3


def kernel(x, Wq, Wo, K_ext, V_ext):
    skv = K_ext.shape[1]
    x2 = x.reshape(SQ, D)
    k3 = K_ext.reshape(skv, HQ, DH)
    v3 = V_ext.reshape(skv, HQ, DH)

    def body(x_ref, wq_ref, wo_ref, k_ref, v_ref, out_ref,
             kbuf, vbuf, outc, statc, copy_sems, send_sems, recv_sems):
        my = lax.axis_index("i")

        barrier = pltpu.get_barrier_semaphore()
        for r in range(N_ROUNDS):
            pl.semaphore_signal(
                barrier, inc=1,
                device_id=(my ^ (1 << r),),
                device_id_type=pl.DeviceIdType.MESH,
            )
        pl.semaphore_wait(barrier, N_ROUNDS)

        def start_head_dma(h, slot):
            pltpu.make_async_copy(
                k_ref.at[:, h, :], kbuf.at[slot], copy_sems.at[slot, 0]
            ).start()
            pltpu.make_async_copy(
                v_ref.at[:, h, :], vbuf.at[slot], copy_sems.at[slot, 1]
            ).start()

        def wait_head_dma(h, slot):
            pltpu.make_async_copy(
                k_ref.at[:, h, :], kbuf.at[slot], copy_sems.at[slot, 0]
            ).wait()
            pltpu.make_async_copy(
                v_ref.at[:, h, :], vbuf.at[slot], copy_sems.at[slot, 1]
            ).wait()

        xq = x_ref[:, :].astype(jnp.bfloat16)
        wq = wq_ref[:, :].astype(jnp.bfloat16)
        q = lax.dot_general(xq, wq, (((1,), (0,)), ((), ())),
                            preferred_element_type=jnp.float32)

        start_head_dma(0, 0)
        start_head_dma(1, 1)
        for h in range(HQ):
            slot = h % 2
            wait_head_dma(h, slot)
            qh = q[:, h * DH:(h + 1) * DH].astype(jnp.bfloat16)
            kh = kbuf[slot].astype(jnp.bfloat16)
            vh = vbuf[slot].astype(jnp.bfloat16)
            st = lax.dot_general(kh, qh, (((1,), (1,)), ((), ())),
                                 preferred_element_type=jnp.float32)
            st = st * SCALE
            mt = jnp.max(st, axis=0, keepdims=True)
            pt = jnp.exp(st - mt)
            lt = jnp.sum(pt, axis=0, keepdims=True)
            ot = lax.dot_general(vh, pt.astype(jnp.bfloat16),
                                 (((0,), (0,)), ((), ())),
                                 preferred_element_type=jnp.float32)
            if h + 2 < HQ:
                start_head_dma(h + 2, slot)
            outc[0, h] = ot.astype(jnp.bfloat16)
            statc[0, h:h + 1, :] = mt
            statc[0, HQ + h:HQ + h + 1, :] = lt

        for r in range(N_ROUNDS):
            partner = my ^ (1 << r)
            s_out = pltpu.make_async_remote_copy(
                src_ref=outc.at[2 * r],
                dst_ref=outc.at[2 * r + 1],
                send_sem=send_sems.at[r, 0],
                recv_sem=recv_sems.at[r, 0],
                device_id=(partner,),
                device_id_type=pl.DeviceIdType.MESH,
            )
            s_st = pltpu.make_async_remote_copy(
                src_ref=statc.at[2 * r],
                dst_ref=statc.at[2 * r + 1],
                send_sem=send_sems.at[r, 1],
                recv_sem=recv_sems.at[r, 1],
                device_id=(partner,),
                device_id_type=pl.DeviceIdType.MESH,
            )
            s_out.start()
            s_st.start()
            s_out.wait()
            s_st.wait()

            m1 = statc[2 * r, 0:HQ, :]
            l1 = statc[2 * r, HQ:2 * HQ, :]
            m2 = statc[2 * r + 1, 0:HQ, :]
            l2 = statc[2 * r + 1, HQ:2 * HQ, :]
            mm = jnp.maximum(m1, m2)
            w1 = jnp.exp(m1 - mm)
            w2 = jnp.exp(m2 - mm)
            lm = w1 * l1 + w2 * l2
            om = (w1[:, None, :] * outc[2 * r].astype(jnp.float32)
                  + w2[:, None, :] * outc[2 * r + 1].astype(jnp.float32))

            if r < N_ROUNDS - 1:
                outc[2 * (r + 1)] = om.astype(jnp.bfloat16)
                statc[2 * (r + 1), 0:HQ, :] = mm
                statc[2 * (r + 1), HQ:2 * HQ, :] = lm
            else:
                attn = om / lm[:, None, :]
                acc = jnp.zeros((SQ, D), jnp.float32)
                for h in range(HQ):
                    ah = attn[h].astype(jnp.bfloat16)
                    woh = wo_ref[h * DH:(h + 1) * DH, :].astype(jnp.bfloat16)
                    acc = acc + lax.dot_general(
                        ah, woh, (((0,), (0,)), ((), ())),
                        preferred_element_type=jnp.float32)
                out_ref[:, :] = acc

    out = pl.pallas_call(
        body,
        out_shape=jax.ShapeDtypeStruct((SQ, D), jnp.float32),
        in_specs=[pl.BlockSpec(memory_space=pltpu.VMEM)] * 3
        + [pl.BlockSpec(memory_space=pltpu.MemorySpace.HBM)] * 2,
        out_specs=pl.BlockSpec(memory_space=pltpu.VMEM),
        scratch_shapes=[
            pltpu.VMEM((2, 4096, DH), jnp.float32),
            pltpu.VMEM((2, 4096, DH), jnp.float32),
            pltpu.VMEM((2 * N_ROUNDS, HQ, DH, SQ), jnp.bfloat16),
            pltpu.VMEM((2 * N_ROUNDS, 2 * HQ, SQ), jnp.float32),
            pltpu.SemaphoreType.DMA((2, 2)),
            pltpu.SemaphoreType.DMA((N_ROUNDS, 2)),
            pltpu.SemaphoreType.DMA((N_ROUNDS, 2)),
        ],
        compiler_params=pltpu.CompilerParams(
            collective_id=0,
            vmem_limit_bytes=100 * 1024 * 1024,
        ),
    )(x2, Wq, Wo, k3, v3)
    return out.reshape(1, SQ, D)


# device time: 52045 ns/iter; 2.1122x vs baseline; 1.1680x over previous
import jax
import jax.numpy as jnp
from jax import lax
from jax.experimental import pallas as pl
from jax.experimental.pallas import tpu as pltpu

N_DEV = 8
HQ = 8
DH = 128
SQ = 256
D = 1024
SCALE = 0.08838834764831843
N_ROUNDS = 3
HHALF = HQ // 2


def kernel(x, Wq, Wo, K_ext, V_ext):
    skv = K_ext.shape[1]
    x2 = x.reshape(SQ, D)
    k3 = K_ext.reshape(skv, HQ, DH)
    v3 = V_ext.reshape(skv, HQ, DH)

    def body(x_ref, wq_ref, wo_ref, k_ref, v_ref, out_ref,
             kbuf, vbuf, outc, statc, copy_sems, send_sems, recv_sems):
        my = lax.axis_index("i")

        barrier = pltpu.get_barrier_semaphore()
        for r in range(N_ROUNDS):
            pl.semaphore_signal(
                barrier, inc=1,
                device_id=(my ^ (1 << r),),
                device_id_type=pl.DeviceIdType.MESH,
            )
        pl.semaphore_wait(barrier, N_ROUNDS)

        def head_dma(h, slot):
            return (
                pltpu.make_async_copy(
                    k_ref.at[:, h, :], kbuf.at[slot], copy_sems.at[slot, 0]
                ),
                pltpu.make_async_copy(
                    v_ref.at[:, h, :], vbuf.at[slot], copy_sems.at[slot, 1]
                ),
            )

        def round_rdmas(r, half):
            partner = my ^ (1 << r)
            hs = HHALF * half
            o = pltpu.make_async_remote_copy(
                src_ref=outc.at[2 * r, hs:hs + HHALF],
                dst_ref=outc.at[2 * r + 1, hs:hs + HHALF],
                send_sem=send_sems.at[r, half, 0],
                recv_sem=recv_sems.at[r, half, 0],
                device_id=(partner,),
                device_id_type=pl.DeviceIdType.MESH,
            )
            rs = 8 * half
            s = pltpu.make_async_remote_copy(
                src_ref=statc.at[2 * r, rs:rs + 8],
                dst_ref=statc.at[2 * r + 1, rs:rs + 8],
                send_sem=send_sems.at[r, half, 1],
                recv_sem=recv_sems.at[r, half, 1],
                device_id=(partner,),
                device_id_type=pl.DeviceIdType.MESH,
            )
            return o, s

        rdmas = {(r, hf): round_rdmas(r, hf)
                 for r in range(N_ROUNDS) for hf in range(2)}

        xq = x_ref[:, :].astype(jnp.bfloat16)
        wq = wq_ref[:, :].astype(jnp.bfloat16)
        q = lax.dot_general(xq, wq, (((1,), (0,)), ((), ())),
                            preferred_element_type=jnp.float32)

        k0, v0 = head_dma(0, 0)
        k0.start(); v0.start()
        k1, v1 = head_dma(1, 1)
        k1.start(); v1.start()

        for h in range(HQ):
            slot = h % 2
            kd, vd = head_dma(h, slot)
            kd.wait(); vd.wait()
            half, i = divmod(h, HHALF)
            qh = q[:, h * DH:(h + 1) * DH].astype(jnp.bfloat16)
            kh = kbuf[slot].astype(jnp.bfloat16)
            vh = vbuf[slot].astype(jnp.bfloat16)
            st = lax.dot_general(kh, qh, (((1,), (1,)), ((), ())),
                                 preferred_element_type=jnp.float32)
            st = st * SCALE
            mt = jnp.max(st, axis=0, keepdims=True)
            pt = jnp.exp(st - mt)
            lt = jnp.sum(pt, axis=0, keepdims=True)
            ot = lax.dot_general(vh, pt.astype(jnp.bfloat16),
                                 (((0,), (0,)), ((), ())),
                                 preferred_element_type=jnp.float32)
            if h + 2 < HQ:
                nkd, nvd = head_dma(h + 2, slot)
                nkd.start(); nvd.start()
            outc[0, h] = ot.astype(jnp.bfloat16)
            rs = 8 * half
            statc[0, rs + i:rs + i + 1, :] = mt
            statc[0, rs + 4 + i:rs + 4 + i + 1, :] = lt
            if h == HHALF - 1 or h == HQ - 1:
                ro, rst = rdmas[(0, half)]
                ro.start(); rst.start()

        def merge(r, half):
            rs = 8 * half
            hs = HHALF * half
            m1 = statc[2 * r, rs:rs + 4, :]
            l1 = statc[2 * r, rs + 4:rs + 8, :]
            m2 = statc[2 * r + 1, rs:rs + 4, :]
            l2 = statc[2 * r + 1, rs + 4:rs + 8, :]
            mm = jnp.maximum(m1, m2)
            w1 = jnp.exp(m1 - mm)
            w2 = jnp.exp(m2 - mm)
            lm = w1 * l1 + w2 * l2
            om = (w1[:, None, :] * outc[2 * r, hs:hs + HHALF].astype(jnp.float32)
                  + w2[:, None, :] * outc[2 * r + 1, hs:hs + HHALF].astype(jnp.float32))
            return om, mm, lm

        def wo_half(om, lm, half):
            attn = om / lm[:, None, :]
            acc = None
            for j in range(HHALF):
                h = HHALF * half + j
                ah = attn[j].astype(jnp.bfloat16)
                woh = wo_ref[h * DH:(h + 1) * DH, :].astype(jnp.bfloat16)
                d = lax.dot_general(ah, woh, (((0,), (0,)), ((), ())),
                                    preferred_element_type=jnp.float32)
                acc = d if acc is None else acc + d
            return acc

        accs = [None, None]
        for r in range(N_ROUNDS):
            for half in range(2):
                ro, rst = rdmas[(r, half)]
                ro.wait_recv(); rst.wait_recv()
                om, mm, lm = merge(r, half)
                rs = 8 * half
                hs = HHALF * half
                if r < N_ROUNDS - 1:
                    outc[2 * (r + 1), hs:hs + HHALF] = om.astype(jnp.bfloat16)
                    statc[2 * (r + 1), rs:rs + 4, :] = mm
                    statc[2 * (r + 1), rs + 4:rs + 8, :] = lm
                    no, ns = rdmas[(r + 1, half)]
                    no.start(); ns.start()
                else:
                    accs[half] = wo_half(om, lm, half)

        out_ref[:, :] = accs[0] + accs[1]

        for r in range(N_ROUNDS):
            for half in range(2):
                ro, rst = rdmas[(r, half)]
                ro.wait_send(); rst.wait_send()

    out = pl.pallas_call(
        body,
        out_shape=jax.ShapeDtypeStruct((SQ, D), jnp.float32),
        in_specs=[pl.BlockSpec(memory_space=pltpu.VMEM)] * 3
        + [pl.BlockSpec(memory_space=pltpu.MemorySpace.HBM)] * 2,
        out_specs=pl.BlockSpec(memory_space=pltpu.VMEM),
        scratch_shapes=[
            pltpu.VMEM((2, 4096, DH), jnp.float32),
            pltpu.VMEM((2, 4096, DH), jnp.float32),
            pltpu.VMEM((2 * N_ROUNDS, HQ, DH, SQ), jnp.bfloat16),
            pltpu.VMEM((2 * N_ROUNDS, 2 * HQ, SQ), jnp.float32),
            pltpu.SemaphoreType.DMA((2, 2)),
            pltpu.SemaphoreType.DMA((N_ROUNDS, 2, 2)),
            pltpu.SemaphoreType.DMA((N_ROUNDS, 2, 2)),
        ],
        compiler_params=pltpu.CompilerParams(
            collective_id=0,
            vmem_limit_bytes=100 * 1024 * 1024,
        ),
    )(x2, Wq, Wo, k3, v3)
    return out.reshape(1, SQ, D)


# device time: 46921 ns/iter; 2.3429x vs baseline; 1.1092x over previous
import jax
import jax.numpy as jnp
from jax import lax
from jax.experimental import pallas as pl
from jax.experimental.pallas import tpu as pltpu

N_DEV = 8
HQ = 8
DH = 128
SQ = 256
D = 1024
SCALE = 0.08838834764831843
N_ROUNDS = 3
NCH = 4
HCH = HQ // NCH


def kernel(x, Wq, Wo, K_ext, V_ext):
    skv = K_ext.shape[1]
    x2 = x.reshape(SQ, D)
    k3 = K_ext.reshape(skv, HQ, DH)
    v3 = V_ext.reshape(skv, HQ, DH)

    def body(x_ref, wq_ref, wo_ref, k_ref, v_ref, out_ref,
             kbuf, vbuf, outc, statc, copy_sems, send_sems, recv_sems):
        my = lax.axis_index("i")

        barrier = pltpu.get_barrier_semaphore()
        for r in range(N_ROUNDS):
            pl.semaphore_signal(
                barrier, inc=1,
                device_id=(my ^ (1 << r),),
                device_id_type=pl.DeviceIdType.MESH,
            )
        pl.semaphore_wait(barrier, N_ROUNDS)

        def head_dma(h, slot):
            return (
                pltpu.make_async_copy(
                    k_ref.at[:, h, :], kbuf.at[slot], copy_sems.at[slot, 0]
                ),
                pltpu.make_async_copy(
                    v_ref.at[:, h, :], vbuf.at[slot], copy_sems.at[slot, 1]
                ),
            )

        def round_rdmas(r, c):
            partner = my ^ (1 << r)
            hs = HCH * c
            o = pltpu.make_async_remote_copy(
                src_ref=outc.at[2 * r, hs:hs + HCH],
                dst_ref=outc.at[2 * r + 1, hs:hs + HCH],
                send_sem=send_sems.at[r, c, 0],
                recv_sem=recv_sems.at[r, c, 0],
                device_id=(partner,),
                device_id_type=pl.DeviceIdType.MESH,
            )
            rs = 4 * c
            s = pltpu.make_async_remote_copy(
                src_ref=statc.at[2 * r, rs:rs + 4],
                dst_ref=statc.at[2 * r + 1, rs:rs + 4],
                send_sem=send_sems.at[r, c, 1],
                recv_sem=recv_sems.at[r, c, 1],
                device_id=(partner,),
                device_id_type=pl.DeviceIdType.MESH,
            )
            return o, s

        rdmas = {(r, c): round_rdmas(r, c)
                 for r in range(N_ROUNDS) for c in range(NCH)}

        def merge(r, c):
            rs = 4 * c
            hs = HCH * c
            m1 = statc[2 * r, rs:rs + 2, :]
            l1 = statc[2 * r, rs + 2:rs + 4, :]
            m2 = statc[2 * r + 1, rs:rs + 2, :]
            l2 = statc[2 * r + 1, rs + 2:rs + 4, :]
            mm = jnp.maximum(m1, m2)
            w1 = jnp.exp(m1 - mm)
            w2 = jnp.exp(m2 - mm)
            lm = w1 * l1 + w2 * l2
            om = (w1[:, None, :] * outc[2 * r, hs:hs + HCH].astype(jnp.float32)
                  + w2[:, None, :] * outc[2 * r + 1, hs:hs + HCH].astype(jnp.float32))
            return om, mm, lm

        accs = [None] * NCH

        def process(r, c):
            ro, rst = rdmas[(r, c)]
            ro.wait_recv(); rst.wait_recv()
            om, mm, lm = merge(r, c)
            if r < N_ROUNDS - 1:
                rs = 4 * c
                hs = HCH * c
                outc[2 * (r + 1), hs:hs + HCH] = om.astype(jnp.bfloat16)
                statc[2 * (r + 1), rs:rs + 2, :] = mm
                statc[2 * (r + 1), rs + 2:rs + 4, :] = lm
                no, ns = rdmas[(r + 1, c)]
                no.start(); ns.start()
            else:
                attn = om / lm[:, None, :]
                acc = None
                for j in range(HCH):
                    h = HCH * c + j
                    ah = attn[j].astype(jnp.bfloat16)
                    woh = wo_ref[h * DH:(h + 1) * DH, :].astype(jnp.bfloat16)
                    d = lax.dot_general(ah, woh, (((0,), (0,)), ((), ())),
                                        preferred_element_type=jnp.float32)
                    acc = d if acc is None else acc + d
                accs[c] = acc

        k0, v0 = head_dma(0, 0)
        k0.start(); v0.start()
        k1, v1 = head_dma(1, 1)
        k1.start(); v1.start()

        xq = x_ref[:, :].astype(jnp.bfloat16)
        wq = wq_ref[:, :].astype(jnp.bfloat16)
        q = lax.dot_general(xq, wq, (((1,), (0,)), ((), ())),
                            preferred_element_type=jnp.float32)

        in_loop = {4: (0, 0), 6: (0, 1)}
        post = [(0, 2), (1, 0), (0, 3), (1, 1), (1, 2), (2, 0),
                (1, 3), (2, 1), (2, 2), (2, 3)]

        for h in range(HQ):
            slot = h % 2
            kd, vd = head_dma(h, slot)
            kd.wait(); vd.wait()
            c, i = divmod(h, HCH)
            qh = q[:, h * DH:(h + 1) * DH].astype(jnp.bfloat16)
            kh = kbuf[slot].astype(jnp.bfloat16)
            vh = vbuf[slot].astype(jnp.bfloat16)
            st = lax.dot_general(kh, qh, (((1,), (1,)), ((), ())),
                                 preferred_element_type=jnp.float32)
            st = st * SCALE
            mt = jnp.max(st, axis=0, keepdims=True)
            pt = jnp.exp((st - mt).astype(jnp.bfloat16))
            lt = jnp.sum(pt, axis=0, keepdims=True, dtype=jnp.float32)
            ot = lax.dot_general(vh, pt,
                                 (((0,), (0,)), ((), ())),
                                 preferred_element_type=jnp.float32)
            if h + 2 < HQ:
                nkd, nvd = head_dma(h + 2, slot)
                nkd.start(); nvd.start()
            outc[0, h] = ot.astype(jnp.bfloat16)
            rs = 4 * c
            statc[0, rs + i:rs + i + 1, :] = mt
            statc[0, rs + 2 + i:rs + 2 + i + 1, :] = lt
            if i == HCH - 1:
                ro, rst = rdmas[(0, c)]
                ro.start(); rst.start()
            if h in in_loop:
                process(*in_loop[h])

        for r, c in post:
            process(r, c)

        out_ref[:, :] = (accs[0] + accs[1]) + (accs[2] + accs[3])

        for r in range(N_ROUNDS):
            for c in range(NCH):
                ro, rst = rdmas[(r, c)]
                ro.wait_send(); rst.wait_send()

    out = pl.pallas_call(
        body,
        out_shape=jax.ShapeDtypeStruct((SQ, D), jnp.float32),
        in_specs=[pl.BlockSpec(memory_space=pltpu.VMEM)] * 3
        + [pl.BlockSpec(memory_space=pltpu.MemorySpace.HBM)] * 2,
        out_specs=pl.BlockSpec(memory_space=pltpu.VMEM),
        scratch_shapes=[
            pltpu.VMEM((2, 4096, DH), jnp.float32),
            pltpu.VMEM((2, 4096, DH), jnp.float32),
            pltpu.VMEM((2 * N_ROUNDS, HQ, DH, SQ), jnp.bfloat16),
            pltpu.VMEM((2 * N_ROUNDS, 2 * HQ, SQ), jnp.float32),
            pltpu.SemaphoreType.DMA((2, 2)),
            pltpu.SemaphoreType.DMA((N_ROUNDS, NCH, 2)),
            pltpu.SemaphoreType.DMA((N_ROUNDS, NCH, 2)),
        ],
        compiler_params=pltpu.CompilerParams(
            collective_id=0,
            vmem_limit_bytes=100 * 1024 * 1024,
        ),
    )(x2, Wq, Wo, k3, v3)
    return out.reshape(1, SQ, D)


# device time: 40965 ns/iter; 2.6835x vs baseline; 1.1454x over previous
import jax
import jax.numpy as jnp
from jax import lax
from jax.experimental import pallas as pl
from jax.experimental.pallas import tpu as pltpu

N_DEV = 8
HQ = 8
DH = 128
SQ = 256
D = 1024
SCALE = 0.08838834764831843
N_ROUNDS = 3
NCH = 4
HCH = HQ // NCH
NSLOT = 4


def kernel(x, Wq, Wo, K_ext, V_ext):
    skv = K_ext.shape[1]
    x2 = x.reshape(SQ, D)
    k3 = K_ext.reshape(skv, HQ, DH)
    v3 = V_ext.reshape(skv, HQ, DH)

    def body(x_ref, wq_ref, wo_ref, k_ref, v_ref, out_ref,
             kbuf, vbuf, outc, statc, copy_sems, send_sems, recv_sems):
        my = lax.axis_index("i")

        barrier = pltpu.get_barrier_semaphore()
        for r in range(N_ROUNDS):
            pl.semaphore_signal(
                barrier, inc=1,
                device_id=(my ^ (1 << r),),
                device_id_type=pl.DeviceIdType.MESH,
            )
        pl.semaphore_wait(barrier, N_ROUNDS)

        def head_dma(h):
            slot = h % NSLOT
            return (
                pltpu.make_async_copy(
                    k_ref.at[:, h, :], kbuf.at[slot], copy_sems.at[slot, 0]
                ),
                pltpu.make_async_copy(
                    v_ref.at[:, h, :], vbuf.at[slot], copy_sems.at[slot, 1]
                ),
            )

        def round_rdmas(r, c):
            partner = my ^ (1 << r)
            hs = HCH * c
            o = pltpu.make_async_remote_copy(
                src_ref=outc.at[2 * r, hs:hs + HCH],
                dst_ref=outc.at[2 * r + 1, hs:hs + HCH],
                send_sem=send_sems.at[r, c, 0],
                recv_sem=recv_sems.at[r, c, 0],
                device_id=(partner,),
                device_id_type=pl.DeviceIdType.MESH,
            )
            rs = 4 * c
            s = pltpu.make_async_remote_copy(
                src_ref=statc.at[2 * r, rs:rs + 4],
                dst_ref=statc.at[2 * r + 1, rs:rs + 4],
                send_sem=send_sems.at[r, c, 1],
                recv_sem=recv_sems.at[r, c, 1],
                device_id=(partner,),
                device_id_type=pl.DeviceIdType.MESH,
            )
            return o, s

        rdmas = {(r, c): round_rdmas(r, c)
                 for r in range(N_ROUNDS) for c in range(NCH)}

        def merge(r, c):
            rs = 4 * c
            hs = HCH * c
            m1 = statc[2 * r, rs:rs + 2, :]
            l1 = statc[2 * r, rs + 2:rs + 4, :]
            m2 = statc[2 * r + 1, rs:rs + 2, :]
            l2 = statc[2 * r + 1, rs + 2:rs + 4, :]
            mm = jnp.maximum(m1, m2)
            w1 = jnp.exp(m1 - mm)
            w2 = jnp.exp(m2 - mm)
            lm = w1 * l1 + w2 * l2
            om = (w1[:, None, :] * outc[2 * r, hs:hs + HCH].astype(jnp.float32)
                  + w2[:, None, :] * outc[2 * r + 1, hs:hs + HCH].astype(jnp.float32))
            return om, mm, lm

        accs = [None] * NCH

        def process(r, c):
            ro, rst = rdmas[(r, c)]
            ro.wait_recv(); rst.wait_recv()
            om, mm, lm = merge(r, c)
            if r < N_ROUNDS - 1:
                rs = 4 * c
                hs = HCH * c
                outc[2 * (r + 1), hs:hs + HCH] = om.astype(jnp.bfloat16)
                statc[2 * (r + 1), rs:rs + 2, :] = mm
                statc[2 * (r + 1), rs + 2:rs + 4, :] = lm
                no, ns = rdmas[(r + 1, c)]
                no.start(); ns.start()
            else:
                attn = om / lm[:, None, :]
                acc = None
                for j in range(HCH):
                    h = HCH * c + j
                    ah = attn[j].astype(jnp.bfloat16)
                    woh = wo_ref[h * DH:(h + 1) * DH, :].astype(jnp.bfloat16)
                    d = lax.dot_general(ah, woh, (((0,), (0,)), ((), ())),
                                        preferred_element_type=jnp.float32)
                    acc = d if acc is None else acc + d
                accs[c] = acc

        for h0 in range(3):
            kd, vd = head_dma(h0)
            kd.start(); vd.start()

        xq = x_ref[:, :].astype(jnp.bfloat16)
        wq = wq_ref[:, :].astype(jnp.bfloat16)
        q = lax.dot_general(xq, wq, (((1,), (0,)), ((), ())),
                            preferred_element_type=jnp.float32)
        qs = (q * SCALE).astype(jnp.bfloat16)

        def qk(h):
            kd, vd = head_dma(h)
            kd.wait(); vd.wait()
            kh = kbuf[h % NSLOT].astype(jnp.bfloat16)
            qh = qs[:, h * DH:(h + 1) * DH]
            return lax.dot_general(kh, qh, (((1,), (1,)), ((), ())),
                                   preferred_element_type=jnp.float32)

        in_loop = {4: (0, 0), 6: (0, 1), 7: (1, 0)}
        post = [(0, 2), (1, 1), (2, 0), (0, 3), (1, 2),
                (2, 1), (1, 3), (2, 2), (2, 3)]

        st_cur = qk(0)
        for h in range(HQ):
            st_next = qk(h + 1) if h + 1 < HQ else None
            c, i = divmod(h, HCH)
            mt = jnp.max(st_cur, axis=0, keepdims=True)
            pt = jnp.exp((st_cur - mt).astype(jnp.bfloat16))
            lt = jnp.sum(pt, axis=0, keepdims=True, dtype=jnp.float32)
            vh = vbuf[h % NSLOT].astype(jnp.bfloat16)
            ot = lax.dot_general(vh, pt, (((0,), (0,)), ((), ())),
                                 preferred_element_type=jnp.float32)
            if h + 3 < HQ:
                nkd, nvd = head_dma(h + 3)
                nkd.start(); nvd.start()
            outc[0, h] = ot.astype(jnp.bfloat16)
            rs = 4 * c
            statc[0, rs + i:rs + i + 1, :] = mt
            statc[0, rs + 2 + i:rs + 2 + i + 1, :] = lt
            if i == HCH - 1:
                ro, rst = rdmas[(0, c)]
                ro.start(); rst.start()
            if h in in_loop:
                process(*in_loop[h])
            st_cur = st_next

        for r, c in post:
            process(r, c)

        out_ref[:, :] = (accs[0] + accs[1]) + (accs[2] + accs[3])

        for r in range(N_ROUNDS):
            for c in range(NCH):
                ro, rst = rdmas[(r, c)]
                ro.wait_send(); rst.wait_send()

    out = pl.pallas_call(
        body,
        out_shape=jax.ShapeDtypeStruct((SQ, D), jnp.float32),
        in_specs=[pl.BlockSpec(memory_space=pltpu.VMEM)] * 3
        + [pl.BlockSpec(memory_space=pltpu.MemorySpace.HBM)] * 2,
        out_specs=pl.BlockSpec(memory_space=pltpu.VMEM),
        scratch_shapes=[
            pltpu.VMEM((NSLOT, 4096, DH), jnp.float32),
            pltpu.VMEM((NSLOT, 4096, DH), jnp.float32),
            pltpu.VMEM((2 * N_ROUNDS, HQ, DH, SQ), jnp.bfloat16),
            pltpu.VMEM((2 * N_ROUNDS, 2 * HQ, SQ), jnp.float32),
            pltpu.SemaphoreType.DMA((NSLOT, 2)),
            pltpu.SemaphoreType.DMA((N_ROUNDS, NCH, 2)),
            pltpu.SemaphoreType.DMA((N_ROUNDS, NCH, 2)),
        ],
        compiler_params=pltpu.CompilerParams(
            collective_id=0,
            vmem_limit_bytes=100 * 1024 * 1024,
        ),
    )(x2, Wq, Wo, k3, v3)
    return out.reshape(1, SQ, D)
